# baseline (speedup 1.0000x reference)
import functools

import jax
import jax.numpy as jnp
from jax import lax
from jax.experimental import pallas as pl
from jax.experimental.pallas import tpu as pltpu
from jax.experimental.pallas import tpu_sc as plsc

NU, NV, D, DS = 25000, 25000, 128, 64
S, NNZ, E = 5, 80000, 400000
HG, HS, HE, NB, NC = 160, 64, 64, 3, 5
H32 = HG // S

NUP = 25088
NW = 32
NTILE = 16
ACC_ROWS = 25088
RPT = ACC_ROWS // NTILE
ZROWS = 196
CH = 128
NPHASE = 2 * S


def _enc_matmul(x4, w_enc_bd):
    n4 = x4.shape[0]
    r4 = 784

    def body(x_ref, w_ref, o_ref):
        xb = x_ref[...]
        for i in range(S):
            o_ref[i] = jnp.dot(xb, w_ref[i], preferred_element_type=jnp.float32)

    return pl.pallas_call(
        body,
        grid=(n4 // r4,),
        in_specs=[
            pl.BlockSpec((r4, 4 * D), lambda g: (g, 0)),
            pl.BlockSpec((S, 4 * D, 128), lambda g: (0, 0, 0)),
        ],
        out_specs=pl.BlockSpec((S, r4, 128), lambda g: (0, g, 0)),
        out_shape=jax.ShapeDtypeStruct((S, n4, 128), jnp.float32),
    )(x4, w_enc_bd)


NNZP = 81920
CPW = NNZP // (NW * CH)


def _seg_sum_sc(tmp_u, tmp_v, sup_row, sup_col, sup_val):
    mesh = plsc.VectorSubcoreMesh(core_axis_name="c", subcore_axis_name="s")

    @functools.partial(
        pl.kernel,
        mesh=mesh,
        out_type=jax.ShapeDtypeStruct((2, NPHASE, ACC_ROWS, H32), jnp.float32),
        compiler_params=pltpu.CompilerParams(use_tc_tiling_on_sc=False),
        scratch_types=[
            pltpu.VMEM((RPT, H32), jnp.float32),
            pltpu.VMEM((CPW, CH), jnp.int32),
            pltpu.VMEM((CPW, CH), jnp.int32),
            pltpu.VMEM((CPW, CH), jnp.float32),
            pltpu.VMEM((CH, H32), jnp.float32),
            pltpu.VMEM((CH, H32), jnp.float32),
            pltpu.VMEM((CH, H32), jnp.float32),
            pltpu.VMEM((CH, H32), jnp.float32),
            pltpu.VMEM_SHARED((ACC_ROWS, H32), jnp.float32),
            pltpu.SemaphoreType.DMA, pltpu.SemaphoreType.DMA,
            pltpu.SemaphoreType.DMA, pltpu.SemaphoreType.DMA,
            pltpu.SemaphoreType.DMA, pltpu.SemaphoreType.DMA,
            pltpu.SemaphoreType.DMA, pltpu.SemaphoreType.DMA,
        ],
    )
    def k(tu_hbm, tv_hbm, row_hbm, col_hbm, val_hbm, out_hbm,
          zbuf, gidx, sidx, vals, rows0, rows1, rows2, rows3, acc,
          sem0, sem1, sem2, sem3, scs0, scs1, scs2, scs3):
        cid = lax.axis_index("c")
        sid = lax.axis_index("s")
        wid = sid * 2 + cid

        def zb(zr, carry):
            zbuf[zr, pl.ds(0, 16)] = jnp.zeros((16,), jnp.float32)
            zbuf[zr, pl.ds(16, 16)] = jnp.zeros((16,), jnp.float32)
            return carry
        lax.fori_loop(0, RPT, zb, 0)

        base_r = sid * RPT
        c0 = wid * CPW
        for p in range(NPHASE):
            i = p % S
            user_side = p < S
            tab = tv_hbm if user_side else tu_hbm
            gsrc = col_hbm if user_side else row_hbm
            ssrc = row_hbm if user_side else col_hbm

            pltpu.sync_copy(zbuf, acc.at[pl.ds(base_r, RPT)])
            pltpu.sync_copy(gsrc.at[i, pl.ds(c0, CPW), :], gidx)
            pltpu.sync_copy(ssrc.at[i, pl.ds(c0, CPW), :], sidx)
            pltpu.sync_copy(val_hbm.at[i, pl.ds(c0, CPW), :], vals)

            def off(c, carry):
                for g in range(CH // 16):
                    gidx[c, pl.ds(g * 16, 16)] = (
                        gidx[c, pl.ds(g * 16, 16)] + (i * NUP))
                return carry
            lax.fori_loop(0, CPW, off, 0)
            plsc.subcore_barrier()

            slots = ((rows0, sem0, scs0), (rows1, sem1, scs1),
                     (rows2, sem2, scs2), (rows3, sem3, scs3))

            def issue(c, slot):
                rows, sem, scs = slot

                @pl.when(c >= 4)
                def _():
                    pltpu.make_async_copy(
                        rows, acc.at[pl.ds(0, CH)], scs).wait()
                pltpu.async_copy(tab.at[gidx.at[c]], rows, sem)

            def drain(c, slot):
                rows, sem, scs = slot
                pltpu.make_async_copy(tab.at[pl.ds(0, CH)], rows, sem).wait()

                def mul(g, c2):
                    vv = vals[c, pl.ds(g * 16, 16)]
                    for t in range(16):
                        e = g * 16 + t
                        sv = vv[t]
                        rows[e, pl.ds(0, 16)] = rows[e, pl.ds(0, 16)] * sv
                        rows[e, pl.ds(16, 16)] = rows[e, pl.ds(16, 16)] * sv
                    return c2
                lax.fori_loop(0, CH // 16, mul, 0)
                pltpu.async_copy(rows, acc.at[sidx.at[c]], scs, add=True)

            for q in range(3):
                issue(q, slots[q])

            def step(j4, carry):
                for q in range(4):
                    @pl.when(j4 * 4 + q + 3 < CPW)
                    def _():
                        issue(j4 * 4 + q + 3, slots[(q + 3) % 4])
                    drain(j4 * 4 + q, slots[q])
                return carry
            lax.fori_loop(0, CPW // 4, step, 0)
            for rows_q, _, scs_q in slots:
                pltpu.make_async_copy(
                    rows_q, acc.at[pl.ds(0, CH)], scs_q).wait()
            plsc.subcore_barrier()
            pltpu.sync_copy(
                acc.at[pl.ds(base_r, RPT)],
                out_hbm.at[cid, p, pl.ds(base_r, RPT)])
            plsc.subcore_barrier()

    return k(tmp_u, tmp_v, sup_row, sup_col, sup_val)


def _embed(h128, side4, w1bd, b1t, w2bd, w2sbd, side_sel, wdbd=None):
    n4 = side4.shape[0]
    r4 = 784
    width = 4 * HE if wdbd is None else 4 * NB * HE

    def body(h_ref, side_ref, w1_ref, b1_ref, w2_ref, w2s_ref, *rest):
        o_ref = rest[-1]
        sh = jax.nn.relu(
            jnp.dot(side_ref[...], w1_ref[...],
                    preferred_element_type=jnp.float32) + b1_ref[...])
        acc = jnp.dot(sh, w2s_ref[...], preferred_element_type=jnp.float32)
        for i in range(S):
            g = jax.nn.relu(h_ref[0, i] + h_ref[1, i])
            acc = acc + jnp.dot(g, w2_ref[i],
                                preferred_element_type=jnp.float32)
        if wdbd is None:
            o_ref[...] = acc
        else:
            wd_ref = rest[0]
            o_ref[...] = jnp.dot(acc, wd_ref[...],
                                 preferred_element_type=jnp.float32)

    in_specs = [
        pl.BlockSpec((2, S, r4, 128), lambda g: (0, side_sel, g, 0)),
        pl.BlockSpec((r4, 4 * DS), lambda g: (g, 0)),
        pl.BlockSpec((4 * DS, 4 * HS), lambda g: (0, 0)),
        pl.BlockSpec((1, 4 * HS), lambda g: (0, 0)),
        pl.BlockSpec((S, 128, 4 * HE), lambda g: (0, 0, 0)),
        pl.BlockSpec((4 * HS, 4 * HE), lambda g: (0, 0)),
    ]
    args = [h128, side4, w1bd, b1t, w2bd, w2sbd]
    if wdbd is not None:
        in_specs.append(pl.BlockSpec((4 * HE, width), lambda g: (0, 0)))
        args.append(wdbd)
    return pl.pallas_call(
        body,
        grid=(n4 // r4,),
        in_specs=in_specs,
        out_specs=pl.BlockSpec((r4, width), lambda g: (g, 0)),
        out_shape=jax.ShapeDtypeStruct((n4, width), jnp.float32),
    )(*args)


EP = 401408
NCH_W = EP // (NW * CH)
PW = NB * HE


def _decode_sc(p_tab, v_tab, uidx, vidx, wpad, rpidx):
    mesh = plsc.VectorSubcoreMesh(core_axis_name="c", subcore_axis_name="s")

    @functools.partial(
        pl.kernel,
        mesh=mesh,
        out_type=jax.ShapeDtypeStruct((EP * NC,), jnp.float32),
        compiler_params=pltpu.CompilerParams(use_tc_tiling_on_sc=False,
                                             needs_layout_passes=False),
        scratch_types=[
            pltpu.VMEM((NCH_W, CH), jnp.int32),
            pltpu.VMEM((NCH_W, CH), jnp.int32),
            pltpu.VMEM((CH, PW), jnp.float32), pltpu.VMEM((CH, PW), jnp.float32),
            pltpu.VMEM((CH, HE), jnp.float32), pltpu.VMEM((CH, HE), jnp.float32),
            pltpu.VMEM((CH * 8,), jnp.float32),
            pltpu.VMEM((CH * NC,), jnp.float32),
            pltpu.VMEM((CH * NC,), jnp.float32),
            pltpu.VMEM((CH * NC,), jnp.int32),
            pltpu.VMEM((96,), jnp.float32),
            pltpu.SemaphoreType.DMA, pltpu.SemaphoreType.DMA,
            pltpu.SemaphoreType.DMA, pltpu.SemaphoreType.DMA,
            pltpu.SemaphoreType.DMA, pltpu.SemaphoreType.DMA,
        ],
    )
    def k(p_hbm, v_hbm, uidx_hbm, vidx_hbm, wpad_hbm, rpidx_hbm, out_hbm,
          iu2d, iv2d, pb0, pb1, vb0, vb1, ob, ob5a, ob5b, rpv, wv,
          sp0, sv0, sp1, sv1, so0, so1):
        cid = lax.axis_index("c")
        sid = lax.axis_index("s")
        wid = sid * 2 + cid
        pltpu.sync_copy(wpad_hbm, wv)
        pltpu.sync_copy(rpidx_hbm, rpv)
        pltpu.sync_copy(uidx_hbm.at[pl.ds(wid * NCH_W, NCH_W), :], iu2d)
        pltpu.sync_copy(vidx_hbm.at[pl.ds(wid * NCH_W, NCH_W), :], iv2d)
        wl = [wv[pl.ds(i * 16, 16)] for i in range(NB)]
        wh = [wv[pl.ds(48 + i * 16, 16)] for i in range(NB)]
        ebase = wid * NCH_W * CH
        slots = ((pb0, vb0, sp0, sv0, ob5a, so0),
                 (pb1, vb1, sp1, sv1, ob5b, so1))

        def issue(j, slot):
            pbs, vbs, sp, sv = slot[:4]
            pltpu.async_copy(p_hbm.at[iu2d.at[j]], pbs, sp)
            pltpu.async_copy(v_hbm.at[iv2d.at[j]], vbs, sv)

        def compute(j, j2, slot):
            pbs, vbs, sp, sv, ob5, so = slot
            pltpu.make_async_copy(p_hbm.at[pl.ds(0, CH)], pbs, sp).wait()
            pltpu.make_async_copy(v_hbm.at[pl.ds(0, CH)], vbs, sv).wait()

            def pair(e2, carry):
                outv = jnp.zeros((16,), jnp.float32)
                for par in range(2):
                    e = e2 * 2 + par
                    v4 = [vbs[e, pl.ds(16 * kk, 16)] for kk in range(4)]
                    for i in range(NB):
                        a = pbs[e, pl.ds(i * HE, 16)] * v4[0]
                        for kk in range(1, 4):
                            a = a + pbs[e, pl.ds(i * HE + 16 * kk, 16)] * v4[kk]
                        b = jnp.sum(a)
                        outv = outv + b * (wl[i] if par == 0 else wh[i])
                ob[pl.ds(e2 * 16, 16)] = outv
                return carry
            lax.fori_loop(0, CH // 2, pair, 0)

            @pl.when(j2 > 0)
            def _():
                pltpu.make_async_copy(
                    ob5, out_hbm.at[pl.ds(0, CH * NC)], so).wait()

            def repack(g, carry):
                iv = rpv[pl.ds(g * 16, 16)]
                ob5[pl.ds(g * 16, 16)] = plsc.load_gather(ob, [iv])
                return carry
            lax.fori_loop(0, CH * NC // 16, repack, 0)
            pltpu.async_copy(
                ob5, out_hbm.at[pl.ds((ebase + j * CH) * NC, CH * NC)], so)

        issue(0, slots[0])

        def step(j2, carry):
            issue(2 * j2 + 1, slots[1])
            compute(2 * j2, j2, slots[0])

            @pl.when(j2 < NCH_W // 2 - 1)
            def _():
                issue(2 * j2 + 2, slots[0])
            compute(2 * j2 + 1, j2, slots[1])
            return carry
        lax.fori_loop(0, NCH_W // 2, step, 0)
        for slot in slots:
            pltpu.make_async_copy(
                slot[4], out_hbm.at[pl.ds(0, CH * NC)], slot[5]).wait()

    return k(p_tab, v_tab, uidx, vidx, wpad, rpidx)


def _format_out(out5_rows):
    eb = 4096
    rb = eb * NC // 128

    def body(x_ref, o_ref):
        o_ref[...] = x_ref[...].reshape(eb, NC)

    return pl.pallas_call(
        body,
        grid=(EP // eb,),
        in_specs=[pl.BlockSpec((rb, 128), lambda g: (g, 0))],
        out_specs=pl.BlockSpec((eb, NC), lambda g: (g, 0)),
        out_shape=jax.ShapeDtypeStruct((EP, NC), jnp.float32),
    )(out5_rows)


def _bd4(w):
    a, b = w.shape
    out = jnp.zeros((4 * a, 4 * b), jnp.float32)
    for q in range(4):
        out = out.at[q * a:(q + 1) * a, q * b:(q + 1) * b].set(w)
    return out


def kernel(user_inputs, item_inputs, user_side_inputs, item_side_inputs,
           sup_row, sup_col, sup_val, user_edge_idx, item_edge_idx,
           W_enc, W1u, b1u, W1v, b1v, W2u, W2v, W_dec, W_cls):
    sup_row = sup_row.astype(jnp.int32)
    sup_col = sup_col.astype(jnp.int32)
    user_edge_idx = user_edge_idx.astype(jnp.int32)
    item_edge_idx = item_edge_idx.astype(jnp.int32)

    padn = lambda a: jnp.pad(a, ((0, NUP - NU), (0, 0)))
    wencbd = jnp.stack([_bd4(W_enc[i]) for i in range(S)])
    tmp_u = _enc_matmul(padn(user_inputs).reshape(NUP // 4, 4 * D),
                        wencbd).reshape(S * NUP, H32)
    tmp_v = _enc_matmul(padn(item_inputs).reshape(NUP // 4, 4 * D),
                        wencbd).reshape(S * NUP, H32)

    pad3 = lambda a: jnp.pad(a, ((0, 0), (0, NNZP - NNZ))).reshape(S, -1, CH)
    h = _seg_sum_sc(tmp_u, tmp_v, pad3(sup_row), pad3(sup_col),
                    pad3(sup_val))
    h128 = h.reshape(2, NPHASE, ACC_ROWS // 4, 128)

    wdf = jnp.transpose(W_dec, (1, 0, 2)).reshape(HE, NB * HE)
    p_tab = _embed(
        h128, padn(user_side_inputs).reshape(NUP // 4, 4 * DS),
        _bd4(W1u), jnp.tile(b1u, 4).reshape(1, -1),
        jnp.stack([_bd4(W2u[i * H32:(i + 1) * H32]) for i in range(S)]),
        _bd4(W2u[HG:]), 0, wdbd=_bd4(wdf)).reshape(NUP, PW)
    item_embed = _embed(
        h128, padn(item_side_inputs).reshape(NUP // 4, 4 * DS),
        _bd4(W1v), jnp.tile(b1v, 4).reshape(1, -1),
        jnp.stack([_bd4(W2v[i * H32:(i + 1) * H32]) for i in range(S)]),
        _bd4(W2v[HG:]), 1).reshape(NUP, HE)

    wpad = (jnp.zeros((6, 16), jnp.float32)
            .at[:NB, :NC].set(W_cls)
            .at[NB:, 8:8 + NC].set(W_cls)
            .reshape(-1))
    uidx = jnp.pad(user_edge_idx, (0, EP - E)).reshape(-1, CH)
    vidx = jnp.pad(item_edge_idx, (0, EP - E)).reshape(-1, CH)
    pos = jnp.arange(CH * NC, dtype=jnp.int32)
    rpidx = (pos // NC) * 8 + pos % NC
    out5 = _decode_sc(p_tab, item_embed, uidx, vidx, wpad, rpidx)
    return out5.reshape(EP, NC)[:E]


# speedup vs baseline: 1.0008x; 1.0008x over previous
import functools

import jax
import jax.numpy as jnp
from jax import lax
from jax.experimental import pallas as pl
from jax.experimental.pallas import tpu as pltpu
from jax.experimental.pallas import tpu_sc as plsc

NU, NV, D, DS = 25000, 25000, 128, 64
S, NNZ, E = 5, 80000, 400000
HG, HS, HE, NB, NC = 160, 64, 64, 3, 5
H32 = HG // S

NUP = 25088
NW = 32
NTILE = 16
ACC_ROWS = 25088
RPT = ACC_ROWS // NTILE
ZROWS = 196
CH = 128
NPHASE = 2 * S


def _enc_matmul(x4, w_enc_bd):
    n4 = x4.shape[0]
    r4 = 784

    def body(x_ref, w_ref, o_ref):
        xb = x_ref[...]
        for i in range(S):
            o_ref[i] = jnp.dot(xb, w_ref[i], preferred_element_type=jnp.float32)

    return pl.pallas_call(
        body,
        grid=(n4 // r4,),
        in_specs=[
            pl.BlockSpec((r4, 4 * D), lambda g: (g, 0)),
            pl.BlockSpec((S, 4 * D, 128), lambda g: (0, 0, 0)),
        ],
        out_specs=pl.BlockSpec((S, r4, 128), lambda g: (0, g, 0)),
        out_shape=jax.ShapeDtypeStruct((S, n4, 128), jnp.float32),
    )(x4, w_enc_bd)


NNZP = 81920
CPW = NNZP // (NW * CH)


def _seg_sum_sc(tmp_u, tmp_v, sup_row, sup_col, sup_val):
    mesh = plsc.VectorSubcoreMesh(core_axis_name="c", subcore_axis_name="s")

    @functools.partial(
        pl.kernel,
        mesh=mesh,
        out_type=jax.ShapeDtypeStruct((2, NPHASE, ACC_ROWS, H32), jnp.float32),
        compiler_params=pltpu.CompilerParams(use_tc_tiling_on_sc=False),
        scratch_types=[
            pltpu.VMEM((RPT, H32), jnp.float32),
            pltpu.VMEM((CPW, CH), jnp.int32),
            pltpu.VMEM((CPW, CH), jnp.int32),
            pltpu.VMEM((CPW, CH), jnp.float32),
            pltpu.VMEM((CH, H32), jnp.float32),
            pltpu.VMEM((CH, H32), jnp.float32),
            pltpu.VMEM((CH, H32), jnp.float32),
            pltpu.VMEM((CH, H32), jnp.float32),
            pltpu.VMEM_SHARED((ACC_ROWS, H32), jnp.float32),
            pltpu.SemaphoreType.DMA, pltpu.SemaphoreType.DMA,
            pltpu.SemaphoreType.DMA, pltpu.SemaphoreType.DMA,
            pltpu.SemaphoreType.DMA, pltpu.SemaphoreType.DMA,
            pltpu.SemaphoreType.DMA, pltpu.SemaphoreType.DMA,
        ],
    )
    def k(tu_hbm, tv_hbm, row_hbm, col_hbm, val_hbm, out_hbm,
          zbuf, gidx, sidx, vals, rows0, rows1, rows2, rows3, acc,
          sem0, sem1, sem2, sem3, scs0, scs1, scs2, scs3):
        cid = lax.axis_index("c")
        sid = lax.axis_index("s")
        wid = sid * 2 + cid

        def zb(zr, carry):
            zbuf[zr, pl.ds(0, 16)] = jnp.zeros((16,), jnp.float32)
            zbuf[zr, pl.ds(16, 16)] = jnp.zeros((16,), jnp.float32)
            return carry
        lax.fori_loop(0, RPT, zb, 0)

        base_r = sid * RPT
        c0 = wid * CPW
        for p in range(NPHASE):
            i = p % S
            user_side = p < S
            tab = tv_hbm if user_side else tu_hbm
            gsrc = col_hbm if user_side else row_hbm
            ssrc = row_hbm if user_side else col_hbm

            pltpu.sync_copy(zbuf, acc.at[pl.ds(base_r, RPT)])
            pltpu.sync_copy(gsrc.at[i, pl.ds(c0, CPW), :], gidx)
            pltpu.sync_copy(ssrc.at[i, pl.ds(c0, CPW), :], sidx)
            pltpu.sync_copy(val_hbm.at[i, pl.ds(c0, CPW), :], vals)

            def off(c, carry):
                for g in range(CH // 16):
                    gidx[c, pl.ds(g * 16, 16)] = (
                        gidx[c, pl.ds(g * 16, 16)] + (i * NUP))
                return carry
            lax.fori_loop(0, CPW, off, 0)
            plsc.subcore_barrier()

            slots = ((rows0, sem0, scs0), (rows1, sem1, scs1),
                     (rows2, sem2, scs2), (rows3, sem3, scs3))

            def issue(c, slot):
                rows, sem, scs = slot

                @pl.when(c >= 4)
                def _():
                    pltpu.make_async_copy(
                        rows, acc.at[pl.ds(0, CH)], scs).wait()
                pltpu.async_copy(tab.at[gidx.at[c]], rows, sem)

            def drain(c, slot):
                rows, sem, scs = slot
                pltpu.make_async_copy(tab.at[pl.ds(0, CH)], rows, sem).wait()

                def mul(g, c2):
                    vv = vals[c, pl.ds(g * 16, 16)]
                    for t in range(16):
                        e = g * 16 + t
                        sv = vv[t]
                        rows[e, pl.ds(0, 16)] = rows[e, pl.ds(0, 16)] * sv
                        rows[e, pl.ds(16, 16)] = rows[e, pl.ds(16, 16)] * sv
                    return c2
                lax.fori_loop(0, CH // 16, mul, 0)
                pltpu.async_copy(rows, acc.at[sidx.at[c]], scs, add=True)

            for q in range(3):
                issue(q, slots[q])

            def step(j4, carry):
                for q in range(4):
                    @pl.when(j4 * 4 + q + 3 < CPW)
                    def _():
                        issue(j4 * 4 + q + 3, slots[(q + 3) % 4])
                    drain(j4 * 4 + q, slots[q])
                return carry
            lax.fori_loop(0, CPW // 4, step, 0)
            for rows_q, _, scs_q in slots:
                pltpu.make_async_copy(
                    rows_q, acc.at[pl.ds(0, CH)], scs_q).wait()
            plsc.subcore_barrier()
            pltpu.sync_copy(
                acc.at[pl.ds(base_r, RPT)],
                out_hbm.at[cid, p, pl.ds(base_r, RPT)])
            plsc.subcore_barrier()

    return k(tmp_u, tmp_v, sup_row, sup_col, sup_val)


def _embed(h128, side4, w1bd, b1t, w2bd, w2sbd, side_sel, wdbd=None):
    n4 = side4.shape[0]
    r4 = 784
    width = 4 * HE if wdbd is None else 4 * NB * HE

    def body(h_ref, side_ref, w1_ref, b1_ref, w2_ref, w2s_ref, *rest):
        o_ref = rest[-1]
        sh = jax.nn.relu(
            jnp.dot(side_ref[...], w1_ref[...],
                    preferred_element_type=jnp.float32) + b1_ref[...])
        acc = jnp.dot(sh, w2s_ref[...], preferred_element_type=jnp.float32)
        for i in range(S):
            g = jax.nn.relu(h_ref[0, i] + h_ref[1, i])
            acc = acc + jnp.dot(g, w2_ref[i],
                                preferred_element_type=jnp.float32)
        if wdbd is None:
            o_ref[...] = acc
        else:
            wd_ref = rest[0]
            o_ref[...] = jnp.dot(acc, wd_ref[...],
                                 preferred_element_type=jnp.float32)

    in_specs = [
        pl.BlockSpec((2, S, r4, 128), lambda g: (0, side_sel, g, 0)),
        pl.BlockSpec((r4, 4 * DS), lambda g: (g, 0)),
        pl.BlockSpec((4 * DS, 4 * HS), lambda g: (0, 0)),
        pl.BlockSpec((1, 4 * HS), lambda g: (0, 0)),
        pl.BlockSpec((S, 128, 4 * HE), lambda g: (0, 0, 0)),
        pl.BlockSpec((4 * HS, 4 * HE), lambda g: (0, 0)),
    ]
    args = [h128, side4, w1bd, b1t, w2bd, w2sbd]
    if wdbd is not None:
        in_specs.append(pl.BlockSpec((4 * HE, width), lambda g: (0, 0)))
        args.append(wdbd)
    return pl.pallas_call(
        body,
        grid=(n4 // r4,),
        in_specs=in_specs,
        out_specs=pl.BlockSpec((r4, width), lambda g: (g, 0)),
        out_shape=jax.ShapeDtypeStruct((n4, width), jnp.float32),
    )(*args)


EP = 401408
NCH_W = EP // (NW * CH)
PW = NB * HE


def _decode_sc(p_tab, v_tab, uidx, vidx, wpad, rpidx):
    mesh = plsc.VectorSubcoreMesh(core_axis_name="c", subcore_axis_name="s")

    @functools.partial(
        pl.kernel,
        mesh=mesh,
        out_type=jax.ShapeDtypeStruct((EP * NC,), jnp.float32),
        compiler_params=pltpu.CompilerParams(use_tc_tiling_on_sc=False,
                                             needs_layout_passes=False),
        scratch_types=[
            pltpu.VMEM((NCH_W, CH), jnp.int32),
            pltpu.VMEM((NCH_W, CH), jnp.int32),
            pltpu.VMEM((CH, PW), jnp.float32), pltpu.VMEM((CH, PW), jnp.float32),
            pltpu.VMEM((CH, HE), jnp.float32), pltpu.VMEM((CH, HE), jnp.float32),
            pltpu.VMEM((CH * 8,), jnp.float32),
            pltpu.VMEM((CH * NC,), jnp.float32),
            pltpu.VMEM((CH * NC,), jnp.float32),
            pltpu.VMEM((CH * NC,), jnp.int32),
            pltpu.VMEM((96,), jnp.float32),
            pltpu.SemaphoreType.DMA, pltpu.SemaphoreType.DMA,
            pltpu.SemaphoreType.DMA, pltpu.SemaphoreType.DMA,
            pltpu.SemaphoreType.DMA, pltpu.SemaphoreType.DMA,
        ],
    )
    def k(p_hbm, v_hbm, uidx_hbm, vidx_hbm, wpad_hbm, rpidx_hbm, out_hbm,
          iu2d, iv2d, pb0, pb1, vb0, vb1, ob, ob5a, ob5b, rpv, wv,
          sp0, sv0, sp1, sv1, so0, so1):
        cid = lax.axis_index("c")
        sid = lax.axis_index("s")
        wid = sid * 2 + cid
        pltpu.sync_copy(wpad_hbm, wv)
        pltpu.sync_copy(rpidx_hbm, rpv)
        pltpu.sync_copy(uidx_hbm.at[pl.ds(wid * NCH_W, NCH_W), :], iu2d)
        pltpu.sync_copy(vidx_hbm.at[pl.ds(wid * NCH_W, NCH_W), :], iv2d)
        wl = [wv[pl.ds(i * 16, 16)] for i in range(NB)]
        wh = [wv[pl.ds(48 + i * 16, 16)] for i in range(NB)]
        ebase = wid * NCH_W * CH
        slots = ((pb0, vb0, sp0, sv0, ob5a, so0),
                 (pb1, vb1, sp1, sv1, ob5b, so1))

        def issue(j, slot):
            pbs, vbs, sp, sv = slot[:4]
            pltpu.async_copy(p_hbm.at[iu2d.at[j]], pbs, sp)
            pltpu.async_copy(v_hbm.at[iv2d.at[j]], vbs, sv)

        def compute(j, j2, slot):
            pbs, vbs, sp, sv, ob5, so = slot
            pltpu.make_async_copy(p_hbm.at[pl.ds(0, CH)], pbs, sp).wait()
            pltpu.make_async_copy(v_hbm.at[pl.ds(0, CH)], vbs, sv).wait()

            def pair(e2, carry):
                outv = jnp.zeros((16,), jnp.float32)
                for par in range(2):
                    e = e2 * 2 + par
                    v4 = [vbs[e, pl.ds(16 * kk, 16)] for kk in range(4)]
                    for i in range(NB):
                        a = pbs[e, pl.ds(i * HE, 16)] * v4[0]
                        for kk in range(1, 4):
                            a = a + pbs[e, pl.ds(i * HE + 16 * kk, 16)] * v4[kk]
                        b = jnp.sum(a)
                        outv = outv + b * (wl[i] if par == 0 else wh[i])
                ob[pl.ds(e2 * 16, 16)] = outv
                return carry
            lax.fori_loop(0, CH // 2, pair, 0)

            @pl.when(j2 > 0)
            def _():
                pltpu.make_async_copy(
                    ob5, out_hbm.at[pl.ds(0, CH * NC)], so).wait()

            def repack(g, carry):
                iv = rpv[pl.ds(g * 16, 16)]
                ob5[pl.ds(g * 16, 16)] = plsc.load_gather(ob, [iv])
                return carry
            lax.fori_loop(0, CH * NC // 16, repack, 0)
            pltpu.async_copy(
                ob5, out_hbm.at[pl.ds((ebase + j * CH) * NC, CH * NC)], so)

        issue(0, slots[0])

        def step(j2, carry):
            issue(2 * j2 + 1, slots[1])
            compute(2 * j2, j2, slots[0])

            @pl.when(j2 < NCH_W // 2 - 1)
            def _():
                issue(2 * j2 + 2, slots[0])
            compute(2 * j2 + 1, j2, slots[1])
            return carry
        lax.fori_loop(0, NCH_W // 2, step, 0)
        for slot in slots:
            pltpu.make_async_copy(
                slot[4], out_hbm.at[pl.ds(0, CH * NC)], slot[5]).wait()

    return k(p_tab, v_tab, uidx, vidx, wpad, rpidx)


def _bd4(w):
    a, b = w.shape
    out = jnp.zeros((4 * a, 4 * b), jnp.float32)
    for q in range(4):
        out = out.at[q * a:(q + 1) * a, q * b:(q + 1) * b].set(w)
    return out


def kernel(user_inputs, item_inputs, user_side_inputs, item_side_inputs,
           sup_row, sup_col, sup_val, user_edge_idx, item_edge_idx,
           W_enc, W1u, b1u, W1v, b1v, W2u, W2v, W_dec, W_cls):
    sup_row = sup_row.astype(jnp.int32)
    sup_col = sup_col.astype(jnp.int32)
    user_edge_idx = user_edge_idx.astype(jnp.int32)
    item_edge_idx = item_edge_idx.astype(jnp.int32)

    padn = lambda a: jnp.pad(a, ((0, NUP - NU), (0, 0)))
    wencbd = jnp.stack([_bd4(W_enc[i]) for i in range(S)])
    tmp_u = _enc_matmul(padn(user_inputs).reshape(NUP // 4, 4 * D),
                        wencbd).reshape(S * NUP, H32)
    tmp_v = _enc_matmul(padn(item_inputs).reshape(NUP // 4, 4 * D),
                        wencbd).reshape(S * NUP, H32)

    pad3 = lambda a: jnp.pad(a, ((0, 0), (0, NNZP - NNZ))).reshape(S, -1, CH)
    h = _seg_sum_sc(tmp_u, tmp_v, pad3(sup_row), pad3(sup_col),
                    pad3(sup_val))
    h128 = h.reshape(2, NPHASE, ACC_ROWS // 4, 128)

    wdf = jnp.transpose(W_dec, (1, 0, 2)).reshape(HE, NB * HE)
    p_tab = _embed(
        h128, padn(user_side_inputs).reshape(NUP // 4, 4 * DS),
        _bd4(W1u), jnp.tile(b1u, 4).reshape(1, -1),
        jnp.stack([_bd4(W2u[i * H32:(i + 1) * H32]) for i in range(S)]),
        _bd4(W2u[HG:]), 0, wdbd=_bd4(wdf)).reshape(NUP, PW)
    item_embed = _embed(
        h128, padn(item_side_inputs).reshape(NUP // 4, 4 * DS),
        _bd4(W1v), jnp.tile(b1v, 4).reshape(1, -1),
        jnp.stack([_bd4(W2v[i * H32:(i + 1) * H32]) for i in range(S)]),
        _bd4(W2v[HG:]), 1).reshape(NUP, HE)

    wpad = (jnp.zeros((6, 16), jnp.float32)
            .at[:NB, :NC].set(W_cls)
            .at[NB:, 8:8 + NC].set(W_cls)
            .reshape(-1))
    uidx = jnp.pad(user_edge_idx, (0, EP - E)).reshape(-1, CH)
    vidx = jnp.pad(item_edge_idx, (0, EP - E)).reshape(-1, CH)
    pos = jnp.arange(CH * NC, dtype=jnp.int32)
    rpidx = (pos // NC) * 8 + pos % NC
    out5 = _decode_sc(p_tab, item_embed, uidx, vidx, wpad, rpidx)
    return out5.reshape(EP, NC)[:E]
